# Initial kernel scaffold; baseline (speedup 1.0000x reference)
import jax
import jax.numpy as jnp
from jax.experimental import pallas as pl


def kernel(points, colors):
    raise NotImplementedError("write your pallas kernel here")


# speedup vs baseline: 18.1230x; 18.1230x over previous
import functools

import jax
import jax.numpy as jnp
from jax import lax
from jax.experimental import pallas as pl
from jax.experimental.pallas import tpu as pltpu
from jax.experimental.pallas import tpu_sc as plsc

H = W = 512
HW = H * W
BATCH = 8
NPTS = 131072
NC = 2
NS = 16
NW = NC * NS
PPT = HW // NW
RPT = H // NW
CHUNK = 8192
NCHUNK = NPTS // CHUNK
GRP = CHUNK // 16


def _any(m):
    return jnp.max(m.astype(jnp.int32)) > 0


@functools.partial(
    pl.kernel,
    mesh=plsc.VectorSubcoreMesh(core_axis_name="c", subcore_axis_name="s"),
    compiler_params=pltpu.CompilerParams(
        needs_layout_passes=False, use_tc_tiling_on_sc=False),
    out_type=(
        jax.ShapeDtypeStruct((BATCH, H, W), jnp.float32),
        jax.ShapeDtypeStruct((BATCH, 3, H, W), jnp.float32),
        jax.ShapeDtypeStruct((BATCH, H, W), jnp.int32),
    ),
    scratch_types=[
        pltpu.VMEM((CHUNK,), jnp.float32),
        pltpu.VMEM((CHUNK,), jnp.float32),
        pltpu.VMEM((CHUNK,), jnp.float32),
        pltpu.VMEM((PPT,), jnp.float32),
        pltpu.VMEM((PPT,), jnp.int32),
        pltpu.VMEM((PPT,), jnp.int32),
        pltpu.VMEM((PPT,), jnp.float32),
        pltpu.VMEM((RPT, W), jnp.float32),
        pltpu.VMEM((RPT, W), jnp.int32),
        pltpu.VMEM((RPT, W), jnp.float32),
        pltpu.SemaphoreType.DMA,
    ],
)
def _zbuffer_kernel(points, colors, depth_o, img_o, index_o,
                    xb, yb, zc, zbuf, ibuf, gbuf, cbuf, st_f, st_i, st_c, sem):
    wid = lax.axis_index("s") * NC + lax.axis_index("c")
    lo = wid * PPT
    row0 = wid * RPT
    iota = lax.iota(jnp.int32, 16)
    inf16 = jnp.full((16,), jnp.inf, jnp.float32)
    n16 = jnp.full((16,), NPTS, jnp.int32)

    def per_batch(b, carry):
        def init_i(i, c):
            zbuf[pl.ds(i * 16, 16)] = inf16
            ibuf[pl.ds(i * 16, 16)] = n16
            return c
        lax.fori_loop(0, PPT // 16, init_i, 0)

        def per_chunk(ci, c):
            base = ci * CHUNK
            pltpu.sync_copy(points.at[b, 0, pl.ds(base, CHUNK)], xb)
            pltpu.sync_copy(points.at[b, 1, pl.ds(base, CHUNK)], yb)
            pltpu.sync_copy(points.at[b, 2, pl.ds(base, CHUNK)], zc)

            def per_group(g, cc):
                xs = xb[pl.ds(g * 16, 16)]
                ys = yb[pl.ds(g * 16, 16)]
                zs = zc[pl.ds(g * 16, 16)]
                u = jnp.clip((xs * jnp.float32(W)).astype(jnp.int32), 0, W - 1)
                v = jnp.clip((ys * jnp.float32(H)).astype(jnp.int32), 0, H - 1)
                loc = v * W + u - lo
                inr = (loc >= 0) & (loc < PPT)

                @pl.when(_any(inr))
                def _():
                    locs = jnp.where(inr, loc, 0)
                    ids = iota + (base + g * 16)
                    old_z = plsc.load_gather(zbuf, [locs])

                    def zbody(m):
                        plsc.store_scatter(zbuf, [locs], zs, mask=m != 0)
                        cur = plsc.load_gather(zbuf, [locs])
                        return (inr & (zs < cur)).astype(jnp.int32)
                    imp = (inr & (zs < old_z)).astype(jnp.int32)
                    lax.while_loop(_any, zbody, imp)
                    new_z = plsc.load_gather(zbuf, [locs])

                    strict = inr & (new_z < old_z) & (zs == new_z)
                    plsc.store_scatter(ibuf, [locs], n16, mask=strict)

                    elig = inr & (zs == new_z)
                    curid = plsc.load_gather(ibuf, [locs])

                    def ibody(m):
                        plsc.store_scatter(ibuf, [locs], ids, mask=m != 0)
                        cur = plsc.load_gather(ibuf, [locs])
                        return (elig & (ids < cur)).astype(jnp.int32)
                    impi = (elig & (ids < curid)).astype(jnp.int32)
                    lax.while_loop(_any, ibody, impi)
                return cc
            lax.fori_loop(0, GRP, per_group, 0)
            return c
        lax.fori_loop(0, NCHUNK, per_chunk, 0)

        def fin(i, c):
            s = pl.ds(i * 16, 16)
            r = i >> 5
            c0 = (i & 31) * 16
            sc = pl.ds(c0, 16)
            iv = ibuf[s]
            zv = zbuf[s]
            valid = iv < NPTS
            pad = (iota + (lo + i * 16)) & (NPTS - 1)
            gbuf[s] = jnp.where(valid, iv, pad)
            st_f[r, sc] = jnp.where(valid, zv, 0.0)
            st_i[r, sc] = jnp.where(valid, iv, -1)
            return c
        lax.fori_loop(0, PPT // 16, fin, 0)

        pltpu.sync_copy(st_f, depth_o.at[b, pl.ds(row0, RPT)])
        pltpu.sync_copy(st_i, index_o.at[b, pl.ds(row0, RPT)])

        def per_ch(ch, c):
            pltpu.async_copy(colors.at[b, ch].at[gbuf], cbuf, sem).wait()

            def msk(i, cc):
                r = i >> 5
                sc = pl.ds((i & 31) * 16, 16)
                cv = cbuf[pl.ds(i * 16, 16)]
                valid = st_i[r, sc] >= 0
                st_c[r, sc] = jnp.where(valid, cv, 0.0)
                return cc
            lax.fori_loop(0, PPT // 16, msk, 0)
            pltpu.sync_copy(st_c, img_o.at[b, ch, pl.ds(row0, RPT)])
            return c
        lax.fori_loop(0, 3, per_ch, 0)
        return carry
    lax.fori_loop(0, BATCH, per_batch, 0)


def kernel(points, colors):
    return _zbuffer_kernel(points, colors)


# speedup vs baseline: 20.5197x; 1.1322x over previous
import functools

import jax
import jax.numpy as jnp
from jax import lax
from jax.experimental import pallas as pl
from jax.experimental.pallas import tpu as pltpu
from jax.experimental.pallas import tpu_sc as plsc

H = W = 512
HW = H * W
BATCH = 8
NPTS = 131072
NC = 2
NS = 16
NW = NC * NS
PPT = HW // NW
RPT = H // NW
CHUNK = 8192
NCHUNK = NPTS // CHUNK
GRP = CHUNK // 16


def _any(m):
    return plsc.all_reduce_population_count(m != 0)[0] > 0


@functools.partial(
    pl.kernel,
    mesh=plsc.VectorSubcoreMesh(core_axis_name="c", subcore_axis_name="s"),
    compiler_params=pltpu.CompilerParams(
        needs_layout_passes=False, use_tc_tiling_on_sc=False),
    out_type=(
        jax.ShapeDtypeStruct((BATCH, H, W), jnp.float32),
        jax.ShapeDtypeStruct((BATCH, 3, H, W), jnp.float32),
        jax.ShapeDtypeStruct((BATCH, H, W), jnp.int32),
    ),
    scratch_types=[
        pltpu.VMEM((CHUNK,), jnp.float32),
        pltpu.VMEM((CHUNK,), jnp.float32),
        pltpu.VMEM((CHUNK,), jnp.float32),
        pltpu.VMEM((PPT,), jnp.float32),
        pltpu.VMEM((PPT,), jnp.int32),
        pltpu.VMEM((PPT,), jnp.int32),
        pltpu.VMEM((PPT,), jnp.float32),
        pltpu.VMEM((RPT, W), jnp.float32),
        pltpu.VMEM((RPT, W), jnp.int32),
        pltpu.VMEM((RPT, W), jnp.float32),
        pltpu.SemaphoreType.DMA,
    ],
)
def _zbuffer_kernel(points, colors, depth_o, img_o, index_o,
                    xb, yb, zc, zbuf, ibuf, gbuf, cbuf, st_f, st_i, st_c, sem):
    wid = lax.axis_index("s") * NC + lax.axis_index("c")
    lo = wid * PPT
    row0 = wid * RPT
    iota = lax.iota(jnp.int32, 16)
    inf16 = jnp.full((16,), jnp.inf, jnp.float32)
    n16 = jnp.full((16,), NPTS, jnp.int32)

    def per_batch(b, carry):
        def init_i(i, c):
            zbuf[pl.ds(i * 16, 16)] = inf16
            ibuf[pl.ds(i * 16, 16)] = n16
            return c
        lax.fori_loop(0, PPT // 16, init_i, 0)

        def per_chunk(ci, c):
            base = ci * CHUNK
            pltpu.sync_copy(points.at[b, 0, pl.ds(base, CHUNK)], xb)
            pltpu.sync_copy(points.at[b, 1, pl.ds(base, CHUNK)], yb)
            pltpu.sync_copy(points.at[b, 2, pl.ds(base, CHUNK)], zc)

            def per_group(g, cc):
                xs = xb[pl.ds(g * 16, 16)]
                ys = yb[pl.ds(g * 16, 16)]
                zs = zc[pl.ds(g * 16, 16)]
                u = jnp.clip((xs * jnp.float32(W)).astype(jnp.int32), 0, W - 1)
                v = jnp.clip((ys * jnp.float32(H)).astype(jnp.int32), 0, H - 1)
                loc = v * W + u - lo
                inr = (loc >= 0) & (loc < PPT)

                @pl.when(_any(inr))
                def _():
                    locs = jnp.where(inr, loc, 0)
                    ids = iota + (base + g * 16)
                    old_z = plsc.load_gather(zbuf, [locs])

                    def zbody(m):
                        plsc.store_scatter(zbuf, [locs], zs, mask=m != 0)
                        cur = plsc.load_gather(zbuf, [locs])
                        return (inr & (zs < cur)).astype(jnp.int32)
                    imp = (inr & (zs < old_z)).astype(jnp.int32)
                    lax.while_loop(_any, zbody, imp)
                    new_z = plsc.load_gather(zbuf, [locs])

                    strict = inr & (new_z < old_z) & (zs == new_z)
                    plsc.store_scatter(ibuf, [locs], n16, mask=strict)

                    elig = inr & (zs == new_z)
                    curid = plsc.load_gather(ibuf, [locs])

                    def ibody(m):
                        plsc.store_scatter(ibuf, [locs], ids, mask=m != 0)
                        cur = plsc.load_gather(ibuf, [locs])
                        return (elig & (ids < cur)).astype(jnp.int32)
                    impi = (elig & (ids < curid)).astype(jnp.int32)
                    lax.while_loop(_any, ibody, impi)
                return cc
            lax.fori_loop(0, GRP, per_group, 0)
            return c
        lax.fori_loop(0, NCHUNK, per_chunk, 0)

        def fin(i, c):
            s = pl.ds(i * 16, 16)
            r = i >> 5
            c0 = (i & 31) * 16
            sc = pl.ds(c0, 16)
            iv = ibuf[s]
            zv = zbuf[s]
            valid = iv < NPTS
            pad = (iota + (lo + i * 16)) & (NPTS - 1)
            gbuf[s] = jnp.where(valid, iv, pad)
            st_f[r, sc] = jnp.where(valid, zv, 0.0)
            st_i[r, sc] = jnp.where(valid, iv, -1)
            return c
        lax.fori_loop(0, PPT // 16, fin, 0)

        pltpu.sync_copy(st_f, depth_o.at[b, pl.ds(row0, RPT)])
        pltpu.sync_copy(st_i, index_o.at[b, pl.ds(row0, RPT)])

        def per_ch(ch, c):
            pltpu.async_copy(colors.at[b, ch].at[gbuf], cbuf, sem).wait()

            def msk(i, cc):
                r = i >> 5
                sc = pl.ds((i & 31) * 16, 16)
                cv = cbuf[pl.ds(i * 16, 16)]
                valid = st_i[r, sc] >= 0
                st_c[r, sc] = jnp.where(valid, cv, 0.0)
                return cc
            lax.fori_loop(0, PPT // 16, msk, 0)
            pltpu.sync_copy(st_c, img_o.at[b, ch, pl.ds(row0, RPT)])
            return c
        lax.fori_loop(0, 3, per_ch, 0)
        return carry
    lax.fori_loop(0, BATCH, per_batch, 0)


def kernel(points, colors):
    return _zbuffer_kernel(points, colors)


# speedup vs baseline: 28.2311x; 1.3758x over previous
import functools

import jax
import jax.numpy as jnp
from jax import lax
from jax.experimental import pallas as pl
from jax.experimental.pallas import tpu as pltpu
from jax.experimental.pallas import tpu_sc as plsc

H = W = 512
HW = H * W
BATCH = 8
NPTS = 131072
NC = 2
NS = 16
NW = NC * NS
PPT = HW // NW
RPT = H // NW
CHUNK = 8192
NCHUNK = NPTS // CHUNK
GRP = CHUNK // 16


def _any(m):
    return plsc.all_reduce_population_count(m != 0)[0] > 0


@functools.partial(
    pl.kernel,
    mesh=plsc.VectorSubcoreMesh(core_axis_name="c", subcore_axis_name="s"),
    compiler_params=pltpu.CompilerParams(
        needs_layout_passes=False, use_tc_tiling_on_sc=False),
    out_type=(
        jax.ShapeDtypeStruct((BATCH, H, W), jnp.float32),
        jax.ShapeDtypeStruct((BATCH, 3, H, W), jnp.float32),
        jax.ShapeDtypeStruct((BATCH, H, W), jnp.int32),
    ),
    scratch_types=[
        pltpu.VMEM((CHUNK,), jnp.float32),
        pltpu.VMEM((CHUNK,), jnp.float32),
        pltpu.VMEM((CHUNK,), jnp.float32),
        pltpu.VMEM((PPT,), jnp.float32),
        pltpu.VMEM((PPT,), jnp.int32),
        pltpu.VMEM((PPT + 16,), jnp.int32),
        pltpu.VMEM((PPT,), jnp.int32),
        pltpu.VMEM((PPT,), jnp.float32),
        pltpu.VMEM((RPT, W), jnp.float32),
        pltpu.VMEM((RPT, W), jnp.int32),
        pltpu.VMEM((RPT, W), jnp.float32),
        pltpu.SemaphoreType.DMA,
    ],
)
def _zbuffer_kernel(points, colors, depth_o, img_o, index_o,
                    xb, yb, zc, zbuf, ibuf, ddet, gbuf, cbuf,
                    st_f, st_i, st_c, sem):
    wid = lax.axis_index("s") * NC + lax.axis_index("c")
    lo = wid * PPT
    row0 = wid * RPT
    iota = lax.iota(jnp.int32, 16)
    inf16 = jnp.full((16,), jnp.inf, jnp.float32)
    n16 = jnp.full((16,), NPTS, jnp.int32)

    def per_batch(b, carry):
        def init_i(i, c):
            zbuf[pl.ds(i * 16, 16)] = inf16
            ibuf[pl.ds(i * 16, 16)] = n16
            return c
        lax.fori_loop(0, PPT // 16, init_i, 0)

        def per_chunk(ci, c):
            base = ci * CHUNK
            pltpu.sync_copy(points.at[b, 0, pl.ds(base, CHUNK)], xb)
            pltpu.sync_copy(points.at[b, 1, pl.ds(base, CHUNK)], yb)
            pltpu.sync_copy(points.at[b, 2, pl.ds(base, CHUNK)], zc)

            def per_group(g, cc):
                xs = xb[pl.ds(g * 16, 16)]
                ys = yb[pl.ds(g * 16, 16)]
                zs = zc[pl.ds(g * 16, 16)]
                u = jnp.clip((xs * jnp.float32(W)).astype(jnp.int32), 0, W - 1)
                v = jnp.clip((ys * jnp.float32(H)).astype(jnp.int32), 0, H - 1)
                loc = v * W + u - lo
                inr = (loc >= 0) & (loc < PPT)

                @pl.when(_any(inr))
                def _():
                    locs = jnp.where(inr, loc, 0)
                    ids = iota + (base + g * 16)
                    locd = jnp.where(inr, loc, PPT + iota)
                    plsc.store_scatter(ddet, [locd], iota)
                    dup = _any(inr & (plsc.load_gather(ddet, [locd]) != iota))

                    @pl.when(jnp.logical_not(dup))
                    def _():
                        cur_z = plsc.load_gather(zbuf, [locs])
                        cur_i = plsc.load_gather(ibuf, [locs])
                        better = inr & ((zs < cur_z) |
                                        ((zs == cur_z) & (ids < cur_i)))
                        plsc.store_scatter(zbuf, [locs], zs, mask=better)
                        plsc.store_scatter(ibuf, [locs], ids, mask=better)

                    @pl.when(dup)
                    def _():
                        old_z = plsc.load_gather(zbuf, [locs])

                        def zbody(m):
                            plsc.store_scatter(zbuf, [locs], zs, mask=m != 0)
                            cur = plsc.load_gather(zbuf, [locs])
                            return (inr & (zs < cur)).astype(jnp.int32)
                        imp = (inr & (zs < old_z)).astype(jnp.int32)
                        lax.while_loop(_any, zbody, imp)
                        new_z = plsc.load_gather(zbuf, [locs])

                        strict = inr & (new_z < old_z) & (zs == new_z)
                        plsc.store_scatter(ibuf, [locs], n16, mask=strict)

                        elig = inr & (zs == new_z)
                        curid = plsc.load_gather(ibuf, [locs])

                        def ibody(m):
                            plsc.store_scatter(ibuf, [locs], ids, mask=m != 0)
                            cur = plsc.load_gather(ibuf, [locs])
                            return (elig & (ids < cur)).astype(jnp.int32)
                        impi = (elig & (ids < curid)).astype(jnp.int32)
                        lax.while_loop(_any, ibody, impi)
                return cc
            lax.fori_loop(0, GRP, per_group, 0)
            return c
        lax.fori_loop(0, NCHUNK, per_chunk, 0)

        def fin(i, c):
            s = pl.ds(i * 16, 16)
            r = i >> 5
            c0 = (i & 31) * 16
            sc = pl.ds(c0, 16)
            iv = ibuf[s]
            zv = zbuf[s]
            valid = iv < NPTS
            pad = (iota + (lo + i * 16)) & (NPTS - 1)
            gbuf[s] = jnp.where(valid, iv, pad)
            st_f[r, sc] = jnp.where(valid, zv, 0.0)
            st_i[r, sc] = jnp.where(valid, iv, -1)
            return c
        lax.fori_loop(0, PPT // 16, fin, 0)

        pltpu.sync_copy(st_f, depth_o.at[b, pl.ds(row0, RPT)])
        pltpu.sync_copy(st_i, index_o.at[b, pl.ds(row0, RPT)])

        def per_ch(ch, c):
            pltpu.async_copy(colors.at[b, ch].at[gbuf], cbuf, sem).wait()

            def msk(i, cc):
                r = i >> 5
                sc = pl.ds((i & 31) * 16, 16)
                cv = cbuf[pl.ds(i * 16, 16)]
                valid = st_i[r, sc] >= 0
                st_c[r, sc] = jnp.where(valid, cv, 0.0)
                return cc
            lax.fori_loop(0, PPT // 16, msk, 0)
            pltpu.sync_copy(st_c, img_o.at[b, ch, pl.ds(row0, RPT)])
            return c
        lax.fori_loop(0, 3, per_ch, 0)
        return carry
    lax.fori_loop(0, BATCH, per_batch, 0)


def kernel(points, colors):
    return _zbuffer_kernel(points, colors)


# speedup vs baseline: 63.2126x; 2.2391x over previous
import functools

import jax
import jax.numpy as jnp
from jax import lax
from jax.experimental import pallas as pl
from jax.experimental.pallas import tpu as pltpu
from jax.experimental.pallas import tpu_sc as plsc

H = W = 512
HW = H * W
BATCH = 8
NPTS = 131072
NC = 2
NS = 16
NW = NC * NS
NSPLIT = 8
NB_PAR = NW // NSPLIT
NPASS = BATCH // NB_PAR
PPT = HW // NSPLIT
RPT = H // NSPLIT
DDET = 4096
CHUNK = 4096
NCHUNK = NPTS // CHUNK
GRP = CHUNK // 16
BLK = 16
NBLK = RPT // BLK
BPX = BLK * W


def _any(m):
    return plsc.all_reduce_population_count(m != 0)[0] > 0


@functools.partial(
    pl.kernel,
    mesh=plsc.VectorSubcoreMesh(core_axis_name="c", subcore_axis_name="s"),
    compiler_params=pltpu.CompilerParams(
        needs_layout_passes=False, use_tc_tiling_on_sc=False),
    out_type=(
        jax.ShapeDtypeStruct((BATCH, H, W), jnp.float32),
        jax.ShapeDtypeStruct((BATCH, 3, H, W), jnp.float32),
        jax.ShapeDtypeStruct((BATCH, H, W), jnp.int32),
    ),
    scratch_types=[
        pltpu.VMEM((CHUNK,), jnp.float32),
        pltpu.VMEM((CHUNK,), jnp.float32),
        pltpu.VMEM((CHUNK,), jnp.float32),
        pltpu.VMEM((PPT,), jnp.float32),
        pltpu.VMEM((PPT,), jnp.int32),
        pltpu.VMEM((DDET + 16,), jnp.int32),
        pltpu.VMEM((BPX,), jnp.int32),
        pltpu.VMEM((BPX,), jnp.float32),
        pltpu.VMEM((BLK, W), jnp.float32),
        pltpu.VMEM((BLK, W), jnp.int32),
        pltpu.SemaphoreType.DMA,
    ],
)
def _zbuffer_kernel(points, colors, depth_o, img_o, index_o,
                    xb, yb, zc, zbuf, ibuf, ddet, gbuf, cbuf,
                    st_f, st_i, sem):
    wid = lax.axis_index("s") * NC + lax.axis_index("c")
    split = wid & (NSPLIT - 1)
    bgrp = wid >> 3
    lo = split * PPT
    row0 = split * RPT
    iota = lax.iota(jnp.int32, 16)
    inf16 = jnp.full((16,), jnp.inf, jnp.float32)
    n16 = jnp.full((16,), NPTS, jnp.int32)

    def per_pass(p, carry):
        b = bgrp + p * NB_PAR

        def init_i(i, c):
            zbuf[pl.ds(i * 16, 16)] = inf16
            ibuf[pl.ds(i * 16, 16)] = n16
            return c
        lax.fori_loop(0, PPT // 16, init_i, 0)

        def per_chunk(ci, c):
            base = ci * CHUNK
            pltpu.sync_copy(points.at[b, 0, pl.ds(base, CHUNK)], xb)
            pltpu.sync_copy(points.at[b, 1, pl.ds(base, CHUNK)], yb)
            pltpu.sync_copy(points.at[b, 2, pl.ds(base, CHUNK)], zc)

            def per_group(g, cc):
                xs = xb[pl.ds(g * 16, 16)]
                ys = yb[pl.ds(g * 16, 16)]
                zs = zc[pl.ds(g * 16, 16)]
                u = jnp.clip((xs * jnp.float32(W)).astype(jnp.int32), 0, W - 1)
                v = jnp.clip((ys * jnp.float32(H)).astype(jnp.int32), 0, H - 1)
                loc = v * W + u - lo
                inr = (loc >= 0) & (loc < PPT)

                @pl.when(_any(inr))
                def _():
                    locs = jnp.where(inr, loc, 0)
                    ids = iota + (base + g * 16)
                    locd = jnp.where(inr, loc & (DDET - 1), DDET + iota)
                    plsc.store_scatter(ddet, [locd], iota)
                    dup = _any(inr &
                               (plsc.load_gather(ddet, [locd]) != iota))

                    @pl.when(jnp.logical_not(dup))
                    def _():
                        cur_z = plsc.load_gather(zbuf, [locs])
                        cur_i = plsc.load_gather(ibuf, [locs])
                        better = inr & ((zs < cur_z) |
                                        ((zs == cur_z) & (ids < cur_i)))
                        plsc.store_scatter(zbuf, [locs], zs, mask=better)
                        plsc.store_scatter(ibuf, [locs], ids, mask=better)

                    @pl.when(dup)
                    def _():
                        old_z = plsc.load_gather(zbuf, [locs])

                        def zbody(m):
                            plsc.store_scatter(zbuf, [locs], zs, mask=m != 0)
                            cur = plsc.load_gather(zbuf, [locs])
                            return (inr & (zs < cur)).astype(jnp.int32)
                        imp = (inr & (zs < old_z)).astype(jnp.int32)
                        lax.while_loop(_any, zbody, imp)
                        new_z = plsc.load_gather(zbuf, [locs])

                        strict = inr & (new_z < old_z) & (zs == new_z)
                        plsc.store_scatter(ibuf, [locs], n16, mask=strict)

                        elig = inr & (zs == new_z)
                        curid = plsc.load_gather(ibuf, [locs])

                        def ibody(m):
                            plsc.store_scatter(ibuf, [locs], ids, mask=m != 0)
                            cur = plsc.load_gather(ibuf, [locs])
                            return (elig & (ids < cur)).astype(jnp.int32)
                        impi = (elig & (ids < curid)).astype(jnp.int32)
                        lax.while_loop(_any, ibody, impi)
                return cc
            lax.fori_loop(0, GRP, per_group, 0)
            return c
        lax.fori_loop(0, NCHUNK, per_chunk, 0)

        def per_blk(blk, c):
            p0 = blk * BPX

            def fin(i, cc):
                pi = p0 + i * 16
                s = pl.ds(pi, 16)
                r = i >> 5
                sc = pl.ds((i & 31) * 16, 16)
                iv = ibuf[s]
                zv = zbuf[s]
                valid = iv < NPTS
                pad = (iota + (lo + pi)) & (NPTS - 1)
                gbuf[pl.ds(i * 16, 16)] = jnp.where(valid, iv, pad)
                st_f[r, sc] = jnp.where(valid, zv, 0.0)
                st_i[r, sc] = jnp.where(valid, iv, -1)
                return cc
            lax.fori_loop(0, BPX // 16, fin, 0)

            rr = pl.ds(row0 + blk * BLK, BLK)
            pltpu.sync_copy(st_f, depth_o.at[b, rr])
            pltpu.sync_copy(st_i, index_o.at[b, rr])

            def per_ch(ch, cc):
                pltpu.async_copy(colors.at[b, ch].at[gbuf], cbuf, sem).wait()

                def msk(i, ccc):
                    r = i >> 5
                    sc = pl.ds((i & 31) * 16, 16)
                    cv = cbuf[pl.ds(i * 16, 16)]
                    valid = st_i[r, sc] >= 0
                    st_f[r, sc] = jnp.where(valid, cv, 0.0)
                    return ccc
                lax.fori_loop(0, BPX // 16, msk, 0)
                pltpu.sync_copy(st_f, img_o.at[b, ch, rr])
                return cc
            lax.fori_loop(0, 3, per_ch, 0)
            return c
        lax.fori_loop(0, NBLK, per_blk, 0)
        return carry
    lax.fori_loop(0, NPASS, per_pass, 0)


def kernel(points, colors):
    return _zbuffer_kernel(points, colors)


# speedup vs baseline: 107.4818x; 1.7003x over previous
import functools

import jax
import jax.numpy as jnp
from jax import lax
from jax.experimental import pallas as pl
from jax.experimental.pallas import tpu as pltpu
from jax.experimental.pallas import tpu_sc as plsc

H = W = 512
HW = H * W
BATCH = 8
NPTS = 131072
NC = 2
NS = 16
NW = NC * NS
NSPLIT = 8
NB_PAR = NW // NSPLIT
NPASS = BATCH // NB_PAR
PPT = HW // NSPLIT
RPT = H // NSPLIT
DDET = 4096
CHUNK = 4096
NCHUNK = NPTS // CHUNK
GRP = CHUNK // 16
BLK = 16
NBLK = RPT // BLK
BPX = BLK * W


def _any(m):
    return plsc.all_reduce_population_count(m != 0)[0] > 0


@functools.partial(
    pl.kernel,
    mesh=plsc.VectorSubcoreMesh(core_axis_name="c", subcore_axis_name="s"),
    compiler_params=pltpu.CompilerParams(
        needs_layout_passes=False, use_tc_tiling_on_sc=False),
    out_type=(
        jax.ShapeDtypeStruct((BATCH, H, W), jnp.float32),
        jax.ShapeDtypeStruct((BATCH, 3, H, W), jnp.float32),
        jax.ShapeDtypeStruct((BATCH, H, W), jnp.int32),
    ),
    scratch_types=[
        pltpu.VMEM((CHUNK,), jnp.float32),
        pltpu.VMEM((CHUNK,), jnp.float32),
        pltpu.VMEM((CHUNK,), jnp.float32),
        pltpu.VMEM((PPT,), jnp.float32),
        pltpu.VMEM((PPT,), jnp.int32),
        pltpu.VMEM((DDET + 16,), jnp.int32),
        pltpu.VMEM((CHUNK + 16,), jnp.int32),
        pltpu.VMEM((BPX,), jnp.int32),
        pltpu.VMEM((BPX,), jnp.float32),
        pltpu.VMEM((BLK, W), jnp.float32),
        pltpu.VMEM((BLK, W), jnp.int32),
        pltpu.SemaphoreType.DMA,
    ],
)
def _zbuffer_kernel(points, colors, depth_o, img_o, index_o,
                    xb, yb, zc, zbuf, ibuf, ddet, fidx, gbuf, cbuf,
                    st_f, st_i, sem):
    wid = lax.axis_index("s") * NC + lax.axis_index("c")
    split = wid & (NSPLIT - 1)
    bgrp = wid >> 3
    lo = split * PPT
    row0 = split * RPT
    iota = lax.iota(jnp.int32, 16)
    inf16 = jnp.full((16,), jnp.inf, jnp.float32)
    n16 = jnp.full((16,), NPTS, jnp.int32)

    def per_pass(p, carry):
        b = bgrp + p * NB_PAR

        def init_i(i, c):
            zbuf[pl.ds(i * 16, 16)] = inf16
            ibuf[pl.ds(i * 16, 16)] = n16
            return c
        lax.fori_loop(0, PPT // 16, init_i, 0)

        def per_chunk(ci, c):
            base = ci * CHUNK
            pltpu.sync_copy(points.at[b, 0, pl.ds(base, CHUNK)], xb)
            pltpu.sync_copy(points.at[b, 1, pl.ds(base, CHUNK)], yb)
            pltpu.sync_copy(points.at[b, 2, pl.ds(base, CHUNK)], zc)

            def filt(g, cnt):
                ys = yb[pl.ds(g * 16, 16)]
                v = (ys * jnp.float32(H)).astype(jnp.int32)
                keep = (v >> 6) == split
                plsc.store_compressed(fidx.at[pl.ds(cnt, 16)],
                                      iota + g * 16, mask=keep)
                return cnt + plsc.all_reduce_population_count(keep)[0]
            cnt = lax.fori_loop(0, GRP, filt, jnp.int32(0))
            ng = (cnt + 15) >> 4

            def per_group(g, cc):
                am = (iota + g * 16) < cnt
                idxs = fidx[pl.ds(g * 16, 16)] & (CHUNK - 1)
                xs = plsc.load_gather(xb, [idxs])
                ys = plsc.load_gather(yb, [idxs])
                zs = plsc.load_gather(zc, [idxs])
                u = jnp.clip((xs * jnp.float32(W)).astype(jnp.int32), 0, W - 1)
                v = jnp.clip((ys * jnp.float32(H)).astype(jnp.int32), 0, H - 1)
                loc = v * W + u - lo
                locs = jnp.where(am, loc, 0)
                ids = idxs + base
                locd = jnp.where(am, locs & (DDET - 1), DDET + iota)
                plsc.store_scatter(ddet, [locd], iota)
                dup = _any(am & (plsc.load_gather(ddet, [locd]) != iota))

                @pl.when(jnp.logical_not(dup))
                def _():
                    cur_z = plsc.load_gather(zbuf, [locs])
                    cur_i = plsc.load_gather(ibuf, [locs])
                    better = am & ((zs < cur_z) |
                                   ((zs == cur_z) & (ids < cur_i)))
                    plsc.store_scatter(zbuf, [locs], zs, mask=better)
                    plsc.store_scatter(ibuf, [locs], ids, mask=better)

                @pl.when(dup)
                def _():
                    old_z = plsc.load_gather(zbuf, [locs])

                    def zbody(m):
                        plsc.store_scatter(zbuf, [locs], zs, mask=m != 0)
                        cur = plsc.load_gather(zbuf, [locs])
                        return (am & (zs < cur)).astype(jnp.int32)
                    imp = (am & (zs < old_z)).astype(jnp.int32)
                    lax.while_loop(_any, zbody, imp)
                    new_z = plsc.load_gather(zbuf, [locs])

                    strict = am & (new_z < old_z) & (zs == new_z)
                    plsc.store_scatter(ibuf, [locs], n16, mask=strict)

                    elig = am & (zs == new_z)
                    curid = plsc.load_gather(ibuf, [locs])

                    def ibody(m):
                        plsc.store_scatter(ibuf, [locs], ids, mask=m != 0)
                        cur = plsc.load_gather(ibuf, [locs])
                        return (elig & (ids < cur)).astype(jnp.int32)
                    impi = (elig & (ids < curid)).astype(jnp.int32)
                    lax.while_loop(_any, ibody, impi)
                return cc
            lax.fori_loop(0, ng, per_group, 0)
            return c
        lax.fori_loop(0, NCHUNK, per_chunk, 0)

        def per_blk(blk, c):
            p0 = blk * BPX

            def fin(i, cc):
                pi = p0 + i * 16
                s = pl.ds(pi, 16)
                r = i >> 5
                sc = pl.ds((i & 31) * 16, 16)
                iv = ibuf[s]
                zv = zbuf[s]
                valid = iv < NPTS
                pad = (iota + (lo + pi)) & (NPTS - 1)
                gbuf[pl.ds(i * 16, 16)] = jnp.where(valid, iv, pad)
                st_f[r, sc] = jnp.where(valid, zv, 0.0)
                st_i[r, sc] = jnp.where(valid, iv, -1)
                return cc
            lax.fori_loop(0, BPX // 16, fin, 0)

            rr = pl.ds(row0 + blk * BLK, BLK)
            pltpu.sync_copy(st_f, depth_o.at[b, rr])
            pltpu.sync_copy(st_i, index_o.at[b, rr])

            def per_ch(ch, cc):
                pltpu.async_copy(colors.at[b, ch].at[gbuf], cbuf, sem).wait()

                def msk(i, ccc):
                    r = i >> 5
                    sc = pl.ds((i & 31) * 16, 16)
                    cv = cbuf[pl.ds(i * 16, 16)]
                    valid = st_i[r, sc] >= 0
                    st_f[r, sc] = jnp.where(valid, cv, 0.0)
                    return ccc
                lax.fori_loop(0, BPX // 16, msk, 0)
                pltpu.sync_copy(st_f, img_o.at[b, ch, rr])
                return cc
            lax.fori_loop(0, 3, per_ch, 0)
            return c
        lax.fori_loop(0, NBLK, per_blk, 0)
        return carry
    lax.fori_loop(0, NPASS, per_pass, 0)


def kernel(points, colors):
    return _zbuffer_kernel(points, colors)
